# baseline (speedup 1.0000x reference)
import functools

import jax
import jax.numpy as jnp
from jax import lax
from jax.experimental import pallas as pl
from jax.experimental.pallas import tpu as pltpu
from jax.experimental.pallas import tpu_sc as plsc

_NC = 2
_NS = 16
_NW = _NC * _NS
_B = 16384
_D = 32
_BPW = _B // _NW

_mesh = plsc.VectorSubcoreMesh(core_axis_name="c", subcore_axis_name="s")


@functools.partial(
    pl.kernel,
    mesh=_mesh,
    out_type=jax.ShapeDtypeStruct((_D, _B), jnp.float32),
    scratch_types=[
        pltpu.VMEM((_BPW,), jnp.int32),
        pltpu.VMEM((_BPW, 128), jnp.float32),
        pltpu.VMEM((_D, _BPW), jnp.float32),
        pltpu.SemaphoreType.DMA,
        pltpu.SemaphoreType.DMA,
    ],
    compiler_params=pltpu.CompilerParams(needs_layout_passes=False),
)
def _gather(tab4, idx_hbm, out_t, idx_v, rows4_v, cols_v, isem, sem):
    wid = lax.axis_index("s") * _NC + lax.axis_index("c")
    base = wid * _BPW
    pltpu.async_copy(idx_hbm.at[pl.ds(base, _BPW)], idx_v, isem).wait()

    def fetch(g, carry):
        vec = idx_v[pl.ds(g * 16, 16)]
        for j in range(16):
            q = vec[j] >> 2
            pltpu.async_copy(
                tab4.at[pl.ds(q, 1)],
                rows4_v.at[pl.ds(g * 16 + j, 1)],
                sem,
            )
        return carry

    lax.fori_loop(0, _BPW // 16, fetch, 0)
    pltpu.make_async_copy(tab4.at[pl.ds(0, _BPW)], rows4_v, sem).wait()

    lane = lax.iota(jnp.int32, 16)

    def extract(g, carry):
        slot = g * 16 + lane
        vec = idx_v[pl.ds(g * 16, 16)]
        w = (vec & 3) * 32
        for c in range(_D):
            vals = plsc.load_gather(rows4_v, [slot, w + c])
            plsc.store_scatter(cols_v, [jnp.full((16,), c, jnp.int32), slot], vals)
        return carry

    lax.fori_loop(0, _BPW // 16, extract, 0)
    pltpu.sync_copy(cols_v, out_t.at[:, pl.ds(base, _BPW)])


def kernel(x, el):
    return _gather(x.reshape(250000, 128), el.astype(jnp.int32)).T


# speedup vs baseline: 3.5684x; 3.5684x over previous
import functools

import jax
import jax.numpy as jnp
from jax import lax
from jax.experimental import pallas as pl
from jax.experimental.pallas import tpu as pltpu
from jax.experimental.pallas import tpu_sc as plsc

_NC = 2
_NS = 16
_NW = _NC * _NS
_B = 16384
_D = 32
_BPW = _B // _NW
_G = 16

_mesh = plsc.VectorSubcoreMesh(core_axis_name="c", subcore_axis_name="s")


@functools.partial(
    pl.kernel,
    mesh=_mesh,
    out_type=jax.ShapeDtypeStruct((_D, _B), jnp.float32),
    scratch_types=[
        pltpu.VMEM((_BPW,), jnp.int32),
        pltpu.VMEM((_D, _G * 128), jnp.float32),
        pltpu.VMEM((_D, _BPW), jnp.float32),
        pltpu.SemaphoreType.DMA,
        pltpu.SemaphoreType.DMA,
    ],
    compiler_params=pltpu.CompilerParams(needs_layout_passes=False),
)
def _gather(tab_t, idx_hbm, out_t, idx_v, blk_v, cols_v, isem, sem):
    wid = lax.axis_index("s") * _NC + lax.axis_index("c")
    base = wid * _BPW
    pltpu.async_copy(idx_hbm.at[pl.ds(base, _BPW)], idx_v, isem).wait()

    lane = lax.iota(jnp.int32, 16)

    def body(g, carry):
        vec = idx_v[pl.ds(g * _G, _G)]
        for j in range(_G):
            rb = pl.multiple_of((vec[j] >> 7) * 128, 128)
            pltpu.async_copy(
                tab_t.at[:, pl.ds(rb, 128)],
                blk_v.at[:, pl.ds(j * 128, 128)],
                sem,
            )
        pltpu.make_async_copy(tab_t.at[:, pl.ds(0, _G * 128)], blk_v, sem).wait()
        lvec = (vec & 127) + lane * 0
        for j in range(_G):
            pos = j * 128 + lvec[j]
            slot = g * _G + j
            lo = plsc.load_gather(blk_v, [lane, jnp.full((16,), pos, jnp.int32)])
            hi = plsc.load_gather(
                blk_v, [lane + 16, jnp.full((16,), pos, jnp.int32)]
            )
            plsc.store_scatter(
                cols_v, [lane, jnp.full((16,), slot, jnp.int32)], lo
            )
            plsc.store_scatter(
                cols_v, [lane + 16, jnp.full((16,), slot, jnp.int32)], hi
            )
        return carry

    lax.fori_loop(0, _BPW // _G, body, 0)
    pltpu.sync_copy(cols_v, out_t.at[:, pl.ds(base, _BPW)])


def kernel(x, el):
    return _gather(x.T, el.astype(jnp.int32)).T


# speedup vs baseline: 3.8097x; 1.0676x over previous
import functools

import jax
import jax.numpy as jnp
from jax import lax
from jax.experimental import pallas as pl
from jax.experimental.pallas import tpu as pltpu
from jax.experimental.pallas import tpu_sc as plsc

_NC = 2
_NS = 16
_NW = _NC * _NS
_B = 16384
_D = 32
_BPW = _B // _NW
_G = 8
_NG = _BPW // _G

_mesh = plsc.VectorSubcoreMesh(core_axis_name="c", subcore_axis_name="s")


@functools.partial(
    pl.kernel,
    mesh=_mesh,
    out_type=jax.ShapeDtypeStruct((_D, _B), jnp.float32),
    scratch_types=[
        pltpu.VMEM((_BPW,), jnp.int32),
        pltpu.VMEM((_D, _G * 128), jnp.float32),
        pltpu.VMEM((_D, _G * 128), jnp.float32),
        pltpu.VMEM((_D, _BPW), jnp.float32),
        pltpu.SemaphoreType.DMA,
        pltpu.SemaphoreType.DMA,
        pltpu.SemaphoreType.DMA,
    ],
    compiler_params=pltpu.CompilerParams(needs_layout_passes=False),
)
def _gather(tab_t, idx_hbm, out_t, idx_v, blk0, blk1, cols_v, isem, sem0, sem1):
    wid = lax.axis_index("s") * _NC + lax.axis_index("c")
    base = wid * _BPW
    pltpu.async_copy(idx_hbm.at[pl.ds(base, _BPW)], idx_v, isem).wait()

    lane = lax.iota(jnp.int32, 16)

    def fire(vec16, j0, blk, sem):
        for j in range(_G):
            rb = pl.multiple_of((vec16[j0 + j] >> 7) * 128, 128)
            pltpu.async_copy(
                tab_t.at[:, pl.ds(rb, 128)],
                blk.at[:, pl.ds(j * 128, 128)],
                sem,
            )

    def drain(blk, sem):
        pltpu.make_async_copy(tab_t.at[:, pl.ds(0, _G * 128)], blk, sem).wait()

    def extract(g, vec16, j0, blk):
        lvec = vec16 & 127
        for j in range(_G):
            pos = jnp.full((16,), j * 128 + lvec[j0 + j], jnp.int32)
            slot = jnp.full((16,), g * _G + j, jnp.int32)
            lo = plsc.load_gather(blk, [lane, pos])
            hi = plsc.load_gather(blk, [lane + 16, pos])
            plsc.store_scatter(cols_v, [lane, slot], lo)
            plsc.store_scatter(cols_v, [lane + 16, slot], hi)

    fire(idx_v[pl.ds(0, 16)], 0, blk0, sem0)

    def pair(p, carry):
        g0 = p * 2
        vec16 = idx_v[pl.ds(p * 16, 16)]
        fire(vec16, _G, blk1, sem1)
        drain(blk0, sem0)
        extract(g0, vec16, 0, blk0)

        @pl.when(p < _NG // 2 - 1)
        def _():
            fire(idx_v[pl.ds(p * 16 + 16, 16)], 0, blk0, sem0)

        drain(blk1, sem1)
        extract(g0 + 1, vec16, _G, blk1)
        return carry

    lax.fori_loop(0, _NG // 2, pair, 0)
    pltpu.sync_copy(cols_v, out_t.at[:, pl.ds(base, _BPW)])


def kernel(x, el):
    return _gather(x.T, el.astype(jnp.int32)).T
